# Initial kernel scaffold; baseline (speedup 1.0000x reference)
import jax
import jax.numpy as jnp
from jax.experimental import pallas as pl


def kernel(inputs, codebook):
    raise NotImplementedError("write your pallas kernel here")


# speedup vs baseline: 1.0606x; 1.0606x over previous
import functools

import jax
import jax.numpy as jnp
from jax import lax
from jax.experimental import pallas as pl
from jax.experimental.pallas import tpu as pltpu
from jax.experimental.pallas import tpu_sc as plsc

N_TOKENS = 16384
N_CODES = 8192
DIM = 256
COMMIT = 0.25

M_BLK = 256
N_BATCH_BLOCKS = N_TOKENS // M_BLK

L_BLK = 512

NC, NS = 2, 16
NW = NC * NS
ROWS_PER_W = N_TOKENS // NW
CHUNK = 128
N_CHUNKS = ROWS_PER_W // CHUNK


def _argmax_body(xn_ref, cbn_ref, idx_ref):
    xn = xn_ref[...]
    cbn = cbn_ref[...]
    s = lax.dot_general(xn, cbn, (((1,), (1,)), ((), ())),
                        preferred_element_type=jnp.float32)
    m = jnp.max(s, axis=1, keepdims=True)
    col = lax.broadcasted_iota(jnp.int32, s.shape, 1)
    idx = jnp.min(jnp.where(s == m, col, jnp.int32(2147483647)), axis=1)
    idx_ref[0, 0, :] = idx


def _loss_body(x_ref, q_ref, acc_ref):
    d = q_ref[...] - x_ref[...]

    @pl.when(pl.program_id(0) == 0)
    def _():
        acc_ref[0, 0] = 0.0

    acc_ref[0, 0] += jnp.sum(d * d)


def _gather_body(cb_hbm, idx_hbm, out_hbm, idx_v, rows_v, sem):
    wid = lax.axis_index("s") * NC + lax.axis_index("c")
    pltpu.sync_copy(idx_hbm.at[pl.ds(wid * N_CHUNKS, N_CHUNKS)], idx_v)
    for c in range(N_CHUNKS):
        pltpu.async_copy(cb_hbm.at[idx_v.at[c]], rows_v, sem).wait()
        pltpu.sync_copy(
            rows_v, out_hbm.at[pl.ds(wid * ROWS_PER_W + c * CHUNK, CHUNK)])


@functools.lru_cache(maxsize=None)
def _sc_gather():
    return pl.kernel(
        _gather_body,
        out_type=jax.ShapeDtypeStruct((N_TOKENS, DIM), jnp.float32),
        mesh=plsc.VectorSubcoreMesh(core_axis_name="c", subcore_axis_name="s",
                                    num_cores=NC, num_subcores=NS),
        scratch_types=[
            pltpu.VMEM((N_CHUNKS, CHUNK), jnp.int32),
            pltpu.VMEM((CHUNK, DIM), jnp.float32),
            pltpu.SemaphoreType.DMA,
        ],
        compiler_params=pltpu.CompilerParams(use_tc_tiling_on_sc=False),
    )


def kernel(inputs, codebook):
    xn = inputs / jnp.clip(
        jnp.linalg.norm(inputs, axis=1, keepdims=True), 1e-8, None)
    cbn = codebook / jnp.clip(
        jnp.linalg.norm(codebook, axis=1, keepdims=True), 1e-8, None)

    idx3 = pl.pallas_call(
        _argmax_body,
        grid=(N_BATCH_BLOCKS,),
        in_specs=[
            pl.BlockSpec((M_BLK, DIM), lambda i: (i, 0)),
            pl.BlockSpec((N_CODES, DIM), lambda i: (0, 0)),
        ],
        out_specs=pl.BlockSpec((1, 1, M_BLK), lambda i: (i, 0, 0)),
        out_shape=jax.ShapeDtypeStruct((N_BATCH_BLOCKS, 1, M_BLK), jnp.int32),
    )(xn, cbn)
    idx2 = idx3.reshape(NW * N_CHUNKS, CHUNK)

    quantized = _sc_gather()(codebook, idx2)

    partial = pl.pallas_call(
        _loss_body,
        grid=(N_TOKENS // L_BLK,),
        in_specs=[
            pl.BlockSpec((L_BLK, DIM), lambda i: (i, 0)),
            pl.BlockSpec((L_BLK, DIM), lambda i: (i, 0)),
        ],
        out_specs=pl.BlockSpec(
            (1, 1), lambda i: (0, 0), memory_space=pltpu.SMEM),
        out_shape=jax.ShapeDtypeStruct((1, 1), jnp.float32),
    )(inputs, quantized)
    m = partial[0, 0] / (N_TOKENS * DIM)
    loss = m + COMMIT * m
    return quantized, loss


# speedup vs baseline: 1.4013x; 1.3212x over previous
import functools

import jax
import jax.numpy as jnp
from jax import lax
from jax.experimental import pallas as pl
from jax.experimental.pallas import tpu as pltpu
from jax.experimental.pallas import tpu_sc as plsc

N_TOKENS = 16384
N_CODES = 8192
DIM = 256
COMMIT = 0.25

M_BLK = 256
N_BATCH_BLOCKS = N_TOKENS // M_BLK

L_BLK = 512

NC, NS = 2, 16
NW = NC * NS
ROWS_PER_W = N_TOKENS // NW
CHUNK = 128
N_CHUNKS = ROWS_PER_W // CHUNK


def _argmax_body(xn_ref, cbn_ref, idx_ref):
    xn = xn_ref[...]
    cbn = cbn_ref[...]
    s = lax.dot_general(xn, cbn, (((1,), (1,)), ((), ())),
                        preferred_element_type=jnp.float32)
    idx_ref[0, 0, :] = jnp.argmax(s, axis=1).astype(jnp.int32)


def _loss_body(x_ref, q_ref, acc_ref):
    d = q_ref[...] - x_ref[...]

    @pl.when(pl.program_id(0) == 0)
    def _():
        acc_ref[0, 0] = 0.0

    acc_ref[0, 0] += jnp.sum(d * d)


def _gather_body(cb_hbm, idx_hbm, out_hbm, idx_v, rows_v, sem):
    wid = lax.axis_index("s") * NC + lax.axis_index("c")
    pltpu.sync_copy(idx_hbm.at[pl.ds(wid * N_CHUNKS, N_CHUNKS)], idx_v)
    for c in range(N_CHUNKS):
        pltpu.async_copy(cb_hbm.at[idx_v.at[c]], rows_v, sem).wait()
        pltpu.sync_copy(
            rows_v, out_hbm.at[pl.ds(wid * ROWS_PER_W + c * CHUNK, CHUNK)])


@functools.lru_cache(maxsize=None)
def _sc_gather():
    return pl.kernel(
        _gather_body,
        out_type=jax.ShapeDtypeStruct((N_TOKENS, DIM), jnp.float32),
        mesh=plsc.VectorSubcoreMesh(core_axis_name="c", subcore_axis_name="s",
                                    num_cores=NC, num_subcores=NS),
        scratch_types=[
            pltpu.VMEM((N_CHUNKS, CHUNK), jnp.int32),
            pltpu.VMEM((CHUNK, DIM), jnp.float32),
            pltpu.SemaphoreType.DMA,
        ],
        compiler_params=pltpu.CompilerParams(use_tc_tiling_on_sc=False),
    )


def kernel(inputs, codebook):
    xn = inputs / jnp.clip(
        jnp.linalg.norm(inputs, axis=1, keepdims=True), 1e-8, None)
    cbn = codebook / jnp.clip(
        jnp.linalg.norm(codebook, axis=1, keepdims=True), 1e-8, None)

    idx3 = pl.pallas_call(
        _argmax_body,
        grid=(N_BATCH_BLOCKS,),
        in_specs=[
            pl.BlockSpec((M_BLK, DIM), lambda i: (i, 0)),
            pl.BlockSpec((N_CODES, DIM), lambda i: (0, 0)),
        ],
        out_specs=pl.BlockSpec((1, 1, M_BLK), lambda i: (i, 0, 0)),
        out_shape=jax.ShapeDtypeStruct((N_BATCH_BLOCKS, 1, M_BLK), jnp.int32),
    )(xn, cbn)
    idx2 = idx3.reshape(NW * N_CHUNKS, CHUNK)

    quantized = _sc_gather()(codebook, idx2)

    partial = pl.pallas_call(
        _loss_body,
        grid=(N_TOKENS // L_BLK,),
        in_specs=[
            pl.BlockSpec((L_BLK, DIM), lambda i: (i, 0)),
            pl.BlockSpec((L_BLK, DIM), lambda i: (i, 0)),
        ],
        out_specs=pl.BlockSpec(
            (1, 1), lambda i: (0, 0), memory_space=pltpu.SMEM),
        out_shape=jax.ShapeDtypeStruct((1, 1), jnp.float32),
    )(inputs, quantized)
    m = partial[0, 0] / (N_TOKENS * DIM)
    loss = m + COMMIT * m
    return quantized, loss


# speedup vs baseline: 1.5834x; 1.1299x over previous
import functools

import jax
import jax.numpy as jnp
from jax import lax
from jax.experimental import pallas as pl
from jax.experimental.pallas import tpu as pltpu
from jax.experimental.pallas import tpu_sc as plsc

N_TOKENS = 16384
N_CODES = 8192
DIM = 256
COMMIT = 0.25

M_BLK = 512
N_BATCH_BLOCKS = N_TOKENS // M_BLK

L_BLK = 2048

NC, NS = 2, 16
NW = NC * NS
ROWS_PER_W = N_TOKENS // NW
CHUNK = 128
N_CHUNKS = ROWS_PER_W // CHUNK


def _argmax_body(xn_ref, cbn_ref, idx_ref):
    xn = xn_ref[...]
    cbn = cbn_ref[...]
    s = lax.dot_general(xn, cbn, (((1,), (1,)), ((), ())),
                        preferred_element_type=jnp.float32)
    idx_ref[0, 0, :] = jnp.argmax(s, axis=1).astype(jnp.int32)


def _loss_body(x_ref, q_ref, acc_ref):
    d = q_ref[...] - x_ref[...]

    @pl.when(pl.program_id(0) == 0)
    def _():
        acc_ref[0, 0] = 0.0

    acc_ref[0, 0] += jnp.sum(d * d)


def _gather_body(cb_hbm, idx_hbm, out_hbm, idx_v, rows_v, sem):
    wid = lax.axis_index("s") * NC + lax.axis_index("c")
    pltpu.sync_copy(idx_hbm.at[pl.ds(wid * N_CHUNKS, N_CHUNKS)], idx_v)
    for c in range(N_CHUNKS):
        pltpu.async_copy(cb_hbm.at[idx_v.at[c]], rows_v, sem).wait()
        pltpu.sync_copy(
            rows_v, out_hbm.at[pl.ds(wid * ROWS_PER_W + c * CHUNK, CHUNK)])


@functools.lru_cache(maxsize=None)
def _sc_gather():
    return pl.kernel(
        _gather_body,
        out_type=jax.ShapeDtypeStruct((N_TOKENS, DIM), jnp.float32),
        mesh=plsc.VectorSubcoreMesh(core_axis_name="c", subcore_axis_name="s",
                                    num_cores=NC, num_subcores=NS),
        scratch_types=[
            pltpu.VMEM((N_CHUNKS, CHUNK), jnp.int32),
            pltpu.VMEM((CHUNK, DIM), jnp.float32),
            pltpu.SemaphoreType.DMA,
        ],
        compiler_params=pltpu.CompilerParams(use_tc_tiling_on_sc=False),
    )


def kernel(inputs, codebook):
    xn = inputs / jnp.clip(
        jnp.linalg.norm(inputs, axis=1, keepdims=True), 1e-8, None)
    cbn = codebook / jnp.clip(
        jnp.linalg.norm(codebook, axis=1, keepdims=True), 1e-8, None)

    idx3 = pl.pallas_call(
        _argmax_body,
        grid=(N_BATCH_BLOCKS,),
        in_specs=[
            pl.BlockSpec((M_BLK, DIM), lambda i: (i, 0)),
            pl.BlockSpec((N_CODES, DIM), lambda i: (0, 0)),
        ],
        out_specs=pl.BlockSpec((1, 1, M_BLK), lambda i: (i, 0, 0)),
        out_shape=jax.ShapeDtypeStruct((N_BATCH_BLOCKS, 1, M_BLK), jnp.int32),
        compiler_params=pltpu.CompilerParams(vmem_limit_bytes=110 * 2**20),
    )(xn, cbn)
    idx2 = idx3.reshape(NW * N_CHUNKS, CHUNK)

    quantized = _sc_gather()(codebook, idx2)

    partial = pl.pallas_call(
        _loss_body,
        grid=(N_TOKENS // L_BLK,),
        in_specs=[
            pl.BlockSpec((L_BLK, DIM), lambda i: (i, 0)),
            pl.BlockSpec((L_BLK, DIM), lambda i: (i, 0)),
        ],
        out_specs=pl.BlockSpec(
            (1, 1), lambda i: (0, 0), memory_space=pltpu.SMEM),
        out_shape=jax.ShapeDtypeStruct((1, 1), jnp.float32),
    )(inputs, quantized)
    m = partial[0, 0] / (N_TOKENS * DIM)
    loss = m + COMMIT * m
    return quantized, loss


# speedup vs baseline: 1.7399x; 1.0989x over previous
import functools

import jax
import jax.numpy as jnp
from jax import lax
from jax.experimental import pallas as pl
from jax.experimental.pallas import tpu as pltpu
from jax.experimental.pallas import tpu_sc as plsc

N_TOKENS = 16384
N_CODES = 8192
DIM = 256
COMMIT = 0.25

M_BLK = 512
N_BATCH_BLOCKS = N_TOKENS // M_BLK

L_BLK = 2048

NC, NS = 2, 16
NW = NC * NS
ROWS_PER_W = N_TOKENS // NW
CHUNK = 128
N_CHUNKS = ROWS_PER_W // CHUNK


def _argmax_body(x_ref, cbn_ref, idx_ref):
    x = x_ref[...]
    nrm = jnp.clip(jnp.sqrt(jnp.sum(x * x, axis=1, keepdims=True)), 1e-8, None)
    xn = x / nrm
    cbn = cbn_ref[...]
    s = lax.dot_general(xn, cbn, (((1,), (1,)), ((), ())),
                        preferred_element_type=jnp.float32)
    idx_ref[0, 0, :] = jnp.argmax(s, axis=1).astype(jnp.int32)


def _loss_body(x_ref, q_ref, acc_ref):
    d = q_ref[...] - x_ref[...]

    @pl.when(pl.program_id(0) == 0)
    def _():
        acc_ref[0, 0] = 0.0

    acc_ref[0, 0] += jnp.sum(d * d)


def _gather_body(cb_hbm, idx_hbm, out_hbm, idx_v, rows_v, sem):
    wid = lax.axis_index("s") * NC + lax.axis_index("c")
    pltpu.sync_copy(idx_hbm.at[pl.ds(wid * N_CHUNKS, N_CHUNKS)], idx_v)
    for c in range(N_CHUNKS):
        pltpu.async_copy(cb_hbm.at[idx_v.at[c]], rows_v, sem).wait()
        pltpu.sync_copy(
            rows_v, out_hbm.at[pl.ds(wid * ROWS_PER_W + c * CHUNK, CHUNK)])


@functools.lru_cache(maxsize=None)
def _sc_gather():
    return pl.kernel(
        _gather_body,
        out_type=jax.ShapeDtypeStruct((N_TOKENS, DIM), jnp.float32),
        mesh=plsc.VectorSubcoreMesh(core_axis_name="c", subcore_axis_name="s",
                                    num_cores=NC, num_subcores=NS),
        scratch_types=[
            pltpu.VMEM((N_CHUNKS, CHUNK), jnp.int32),
            pltpu.VMEM((CHUNK, DIM), jnp.float32),
            pltpu.SemaphoreType.DMA,
        ],
        compiler_params=pltpu.CompilerParams(use_tc_tiling_on_sc=False),
    )


def kernel(inputs, codebook):
    cbn = codebook / jnp.clip(
        jnp.linalg.norm(codebook, axis=1, keepdims=True), 1e-8, None)

    idx3 = pl.pallas_call(
        _argmax_body,
        grid=(N_BATCH_BLOCKS,),
        in_specs=[
            pl.BlockSpec((M_BLK, DIM), lambda i: (i, 0)),
            pl.BlockSpec((N_CODES, DIM), lambda i: (0, 0)),
        ],
        out_specs=pl.BlockSpec((1, 1, M_BLK), lambda i: (i, 0, 0)),
        out_shape=jax.ShapeDtypeStruct((N_BATCH_BLOCKS, 1, M_BLK), jnp.int32),
        compiler_params=pltpu.CompilerParams(vmem_limit_bytes=110 * 2**20),
    )(inputs, cbn)
    idx2 = idx3.reshape(NW * N_CHUNKS, CHUNK)

    quantized = _sc_gather()(codebook, idx2)

    partial = pl.pallas_call(
        _loss_body,
        grid=(N_TOKENS // L_BLK,),
        in_specs=[
            pl.BlockSpec((L_BLK, DIM), lambda i: (i, 0)),
            pl.BlockSpec((L_BLK, DIM), lambda i: (i, 0)),
        ],
        out_specs=pl.BlockSpec(
            (1, 1), lambda i: (0, 0), memory_space=pltpu.SMEM),
        out_shape=jax.ShapeDtypeStruct((1, 1), jnp.float32),
    )(inputs, quantized)
    m = partial[0, 0] / (N_TOKENS * DIM)
    loss = m + COMMIT * m
    return quantized, loss


# speedup vs baseline: 1.8650x; 1.0719x over previous
import functools

import jax
import jax.numpy as jnp
from jax import lax
from jax.experimental import pallas as pl
from jax.experimental.pallas import tpu as pltpu
from jax.experimental.pallas import tpu_sc as plsc

N_TOKENS = 16384
N_CODES = 8192
DIM = 256
COMMIT = 0.25

M_BLK = 512
N_BATCH_BLOCKS = N_TOKENS // M_BLK

L_BLK = 2048

NC, NS = 2, 16
NW = NC * NS
ROWS_PER_W = N_TOKENS // NW
CHUNK = 128
N_CHUNKS = ROWS_PER_W // CHUNK


def _argmax_body(x_ref, cb_ref, idx_ref, cbn_ref):
    @pl.when(pl.program_id(0) == 0)
    def _():
        cb = cb_ref[...]
        cn = jnp.clip(jnp.sqrt(jnp.sum(cb * cb, axis=1, keepdims=True)),
                      1e-8, None)
        cbn_ref[...] = cb / cn

    x = x_ref[...]
    nrm = jnp.clip(jnp.sqrt(jnp.sum(x * x, axis=1, keepdims=True)), 1e-8, None)
    xn = x / nrm
    s = lax.dot_general(xn, cbn_ref[...], (((1,), (1,)), ((), ())),
                        preferred_element_type=jnp.float32)
    idx_ref[0, 0, :] = jnp.argmax(s, axis=1).astype(jnp.int32)


def _loss_body(x_ref, q_ref, acc_ref):
    d = q_ref[...] - x_ref[...]

    @pl.when(pl.program_id(0) == 0)
    def _():
        acc_ref[0, 0] = 0.0

    acc_ref[0, 0] += jnp.sum(d * d)


def _gather_body(cb_hbm, idx_hbm, out_hbm, idx_v, rows_v, sem):
    wid = lax.axis_index("s") * NC + lax.axis_index("c")
    pltpu.sync_copy(idx_hbm.at[pl.ds(wid * N_CHUNKS, N_CHUNKS)], idx_v)
    for c in range(N_CHUNKS):
        pltpu.async_copy(cb_hbm.at[idx_v.at[c]], rows_v, sem).wait()
        pltpu.sync_copy(
            rows_v, out_hbm.at[pl.ds(wid * ROWS_PER_W + c * CHUNK, CHUNK)])


@functools.lru_cache(maxsize=None)
def _sc_gather():
    return pl.kernel(
        _gather_body,
        out_type=jax.ShapeDtypeStruct((N_TOKENS, DIM), jnp.float32),
        mesh=plsc.VectorSubcoreMesh(core_axis_name="c", subcore_axis_name="s",
                                    num_cores=NC, num_subcores=NS),
        scratch_types=[
            pltpu.VMEM((N_CHUNKS, CHUNK), jnp.int32),
            pltpu.VMEM((CHUNK, DIM), jnp.float32),
            pltpu.SemaphoreType.DMA,
        ],
        compiler_params=pltpu.CompilerParams(use_tc_tiling_on_sc=False),
    )


def kernel(inputs, codebook):
    idx3 = pl.pallas_call(
        _argmax_body,
        grid=(N_BATCH_BLOCKS,),
        in_specs=[
            pl.BlockSpec((M_BLK, DIM), lambda i: (i, 0)),
            pl.BlockSpec((N_CODES, DIM), lambda i: (0, 0)),
        ],
        out_specs=pl.BlockSpec((1, 1, M_BLK), lambda i: (i, 0, 0)),
        out_shape=jax.ShapeDtypeStruct((N_BATCH_BLOCKS, 1, M_BLK), jnp.int32),
        scratch_shapes=[pltpu.VMEM((N_CODES, DIM), jnp.float32)],
        compiler_params=pltpu.CompilerParams(vmem_limit_bytes=110 * 2**20),
    )(inputs, codebook)
    idx2 = idx3.reshape(NW * N_CHUNKS, CHUNK)

    quantized = _sc_gather()(codebook, idx2)

    partial = pl.pallas_call(
        _loss_body,
        grid=(N_TOKENS // L_BLK,),
        in_specs=[
            pl.BlockSpec((L_BLK, DIM), lambda i: (i, 0)),
            pl.BlockSpec((L_BLK, DIM), lambda i: (i, 0)),
        ],
        out_specs=pl.BlockSpec(
            (1, 1), lambda i: (0, 0), memory_space=pltpu.SMEM),
        out_shape=jax.ShapeDtypeStruct((1, 1), jnp.float32),
    )(inputs, quantized)
    m = partial[0, 0] / (N_TOKENS * DIM)
    loss = m + COMMIT * m
    return quantized, loss


# speedup vs baseline: 2.0378x; 1.0927x over previous
import functools

import jax
import jax.numpy as jnp
from jax import lax
from jax.experimental import pallas as pl
from jax.experimental.pallas import tpu as pltpu
from jax.experimental.pallas import tpu_sc as plsc

N_TOKENS = 16384
N_CODES = 8192
DIM = 256
COMMIT = 0.25

M_BLK = 512
N_BATCH_BLOCKS = N_TOKENS // M_BLK

L_BLK = 2048

NC, NS = 2, 16
NW = NC * NS
ROWS_PER_W = N_TOKENS // NW
CHUNK = 128
N_CHUNKS = ROWS_PER_W // CHUNK


def _argmax_body(x_ref, cb_ref, idx_ref, cbn_ref):
    @pl.when(pl.program_id(0) == 0)
    def _():
        cb = cb_ref[...]
        cn = jnp.clip(jnp.sqrt(jnp.sum(cb * cb, axis=1, keepdims=True)),
                      1e-8, None)
        cbn_ref[...] = cb / cn

    x = x_ref[...]
    nrm = jnp.clip(jnp.sqrt(jnp.sum(x * x, axis=1, keepdims=True)), 1e-8, None)
    xn = x / nrm
    s = lax.dot_general(xn, cbn_ref[...], (((1,), (1,)), ((), ())),
                        preferred_element_type=jnp.float32)
    idx_ref[0, 0, :] = jnp.argmax(s, axis=1).astype(jnp.int32)


def _loss_body(x_ref, qv_ref, q_ref, acc_ref):
    q = qv_ref[...].reshape(L_BLK, DIM)
    q_ref[...] = q
    d = q - x_ref[...]

    @pl.when(pl.program_id(0) == 0)
    def _():
        acc_ref[0, 0] = 0.0

    acc_ref[0, 0] += jnp.sum(d * d)


def _gather_body(cb_hbm, idx_hbm, out_hbm, idx_v, rows_v, sem):
    wid = lax.axis_index("s") * NC + lax.axis_index("c")
    pltpu.sync_copy(idx_hbm.at[pl.ds(wid * N_CHUNKS, N_CHUNKS)], idx_v)
    for c in range(N_CHUNKS):
        pltpu.async_copy(cb_hbm.at[idx_v.at[c]], rows_v, sem).wait()
        pltpu.sync_copy(
            rows_v, out_hbm.at[pl.ds(wid * ROWS_PER_W + c * CHUNK, CHUNK)])


@functools.lru_cache(maxsize=None)
def _sc_gather():
    return pl.kernel(
        _gather_body,
        out_type=jax.ShapeDtypeStruct((N_TOKENS, DIM), jnp.float32),
        mesh=plsc.VectorSubcoreMesh(core_axis_name="c", subcore_axis_name="s",
                                    num_cores=NC, num_subcores=NS),
        scratch_types=[
            pltpu.VMEM((N_CHUNKS, CHUNK), jnp.int32),
            pltpu.VMEM((CHUNK, DIM), jnp.float32),
            pltpu.SemaphoreType.DMA,
        ],
        compiler_params=pltpu.CompilerParams(use_tc_tiling_on_sc=False),
    )


def kernel(inputs, codebook):
    idx3 = pl.pallas_call(
        _argmax_body,
        grid=(N_BATCH_BLOCKS,),
        in_specs=[
            pl.BlockSpec((M_BLK, DIM), lambda i: (i, 0)),
            pl.BlockSpec((N_CODES, DIM), lambda i: (0, 0)),
        ],
        out_specs=pl.BlockSpec((1, 1, M_BLK), lambda i: (i, 0, 0)),
        out_shape=jax.ShapeDtypeStruct((N_BATCH_BLOCKS, 1, M_BLK), jnp.int32),
        scratch_shapes=[pltpu.VMEM((N_CODES, DIM), jnp.float32)],
        compiler_params=pltpu.CompilerParams(vmem_limit_bytes=110 * 2**20),
    )(inputs, codebook)
    idx2 = idx3.reshape(NW * N_CHUNKS, CHUNK)

    gathered = _sc_gather()(codebook, idx2)
    qv = gathered.reshape(N_TOKENS // 4, 8, 128)

    quantized, partial = pl.pallas_call(
        _loss_body,
        grid=(N_TOKENS // L_BLK,),
        in_specs=[
            pl.BlockSpec((L_BLK, DIM), lambda i: (i, 0)),
            pl.BlockSpec((L_BLK // 4, 8, 128), lambda i: (i, 0, 0)),
        ],
        out_specs=[
            pl.BlockSpec((L_BLK, DIM), lambda i: (i, 0)),
            pl.BlockSpec((1, 1), lambda i: (0, 0), memory_space=pltpu.SMEM),
        ],
        out_shape=[
            jax.ShapeDtypeStruct((N_TOKENS, DIM), jnp.float32),
            jax.ShapeDtypeStruct((1, 1), jnp.float32),
        ],
    )(inputs, qv)
    m = partial[0, 0] / (N_TOKENS * DIM)
    loss = m + COMMIT * m
    return quantized, loss
